# Initial kernel scaffold; baseline (speedup 1.0000x reference)
import jax
import jax.numpy as jnp
from jax.experimental import pallas as pl


def kernel(input, table):
    raise NotImplementedError("write your pallas kernel here")


# speedup vs baseline: 3.3306x; 3.3306x over previous
import functools

import jax
import jax.numpy as jnp
from jax import lax
from jax.experimental import pallas as pl
from jax.experimental.pallas import tpu as pltpu
from jax.experimental.pallas import tpu_sc as plsc


def kernel(input, table):
    B0, B1 = input.shape
    V, D = table.shape
    B = B0 * B1

    info = plsc.get_sparse_core_info()
    NC, NS = info.num_cores, info.num_subcores
    NW = NC * NS
    CH = 128
    b_per_w = B // NW
    n_ch = b_per_w // CH

    idx = input.reshape(NW, n_ch, CH).astype(jnp.int32)
    mesh = plsc.VectorSubcoreMesh(core_axis_name="c", subcore_axis_name="s")

    @functools.partial(
        pl.kernel,
        out_type=jax.ShapeDtypeStruct((B, D), jnp.float32),
        mesh=mesh,
        scratch_types=[
            pltpu.VMEM((n_ch, CH), jnp.int32),
            pltpu.VMEM((CH, D), jnp.float32),
            pltpu.VMEM((CH, D), jnp.float32),
            pltpu.SemaphoreType.DMA,
            pltpu.SemaphoreType.DMA,
        ],
    )
    def gather_k(table_hbm, idx_hbm, out_hbm, idx_v, rows0, rows1, sem0, sem1):
        wid = lax.axis_index("s") * NC + lax.axis_index("c")
        base = wid * b_per_w
        pltpu.sync_copy(idx_hbm.at[wid], idx_v)

        def wait(buf, sem):
            pltpu.make_async_copy(table_hbm.at[pl.ds(0, CH)], buf, sem).wait()

        pltpu.async_copy(table_hbm.at[idx_v.at[0]], rows0, sem0)

        @pl.loop(0, n_ch, step=2)
        def body(j):
            pltpu.async_copy(table_hbm.at[idx_v.at[j + 1]], rows1, sem1)
            wait(rows0, sem0)
            pltpu.sync_copy(rows0, out_hbm.at[pl.ds(base + j * CH, CH)])
            jf = jnp.minimum(j + 2, n_ch - 1)
            pltpu.async_copy(table_hbm.at[idx_v.at[jf]], rows0, sem0)
            wait(rows1, sem1)
            pltpu.sync_copy(rows1, out_hbm.at[pl.ds(base + (j + 1) * CH, CH)])

        wait(rows0, sem0)

    out = gather_k(table, idx)
    return out.reshape(B0, B1, D)
